# baseline (speedup 1.0000x reference)
import functools

import jax
import jax.numpy as jnp
from jax import lax
from jax.experimental import pallas as pl
from jax.experimental.pallas import tpu as pltpu
from jax.experimental.pallas import tpu_sc as plsc

N = 50000
E = 800000
IN_DIM = 64
H = 108

BR = 1024
NP = 50176
GRID = NP // BR

NSC = 2
NT = 16
CS = NP // 4
ZR = (CS + 16) // NT
CR = CS // NT
EPT = E // NT
CE = 400
NCH = EPT // CE
SEL = CE + 112
GB = 64
SCU = 5
F = 128


def _dense_pre(x, WembT, b_emb, Wp1T, bp1):
    def body(x_ref, we_ref, be_ref, wp_ref, bp_ref, h_ref, g_ref):
        h = jnp.dot(x_ref[...], we_ref[...], preferred_element_type=jnp.float32)
        h = h + be_ref[...]
        h_ref[...] = h
        g = jax.nn.relu(jnp.dot(h, wp_ref[...], preferred_element_type=jnp.float32) + bp_ref[...])
        ones = jnp.ones((BR, 1), jnp.float32)
        zeros = jnp.zeros((BR, F - H - 1), jnp.float32)
        g_ref[...] = jnp.concatenate([g, ones, zeros], axis=1)

    return pl.pallas_call(
        body,
        grid=(GRID,),
        in_specs=[
            pl.BlockSpec((BR, IN_DIM), lambda i: (i, 0)),
            pl.BlockSpec((IN_DIM, H), lambda i: (0, 0)),
            pl.BlockSpec((1, H), lambda i: (0, 0)),
            pl.BlockSpec((H, H), lambda i: (0, 0)),
            pl.BlockSpec((1, H), lambda i: (0, 0)),
        ],
        out_specs=[
            pl.BlockSpec((BR, H), lambda i: (i, 0)),
            pl.BlockSpec((BR, F), lambda i: (i, 0)),
        ],
        out_shape=[
            jax.ShapeDtypeStruct((NP, H), jnp.float32),
            jax.ShapeDtypeStruct((NP, F), jnp.float32),
        ],
    )(x, WembT, b_emb, Wp1T, bp1)


def _node_apply(h, agg, wh, wc, bn):
    cnt = jnp.maximum(agg[:, H:H + 1], 1.0)
    c = agg[:, :H] / cnt
    bundle = (jnp.dot(h, wh, preferred_element_type=jnp.float32)
              + jnp.dot(c, wc, preferred_element_type=jnp.float32) + bn)
    nrm = jnp.maximum(jnp.sqrt(jnp.sum(bundle * bundle, axis=1, keepdims=True)), 1e-12)
    return h + jax.nn.relu(bundle / nrm)


def _dense_mid(h0, agg, WnhT, WncT, bn1, WpoT, bpo):
    def body(h_ref, a_ref, wh_ref, wc_ref, bn_ref, wp_ref, bp_ref, h1_ref, g_ref):
        h1 = _node_apply(h_ref[...], a_ref[...], wh_ref[...], wc_ref[...], bn_ref[...])
        h1_ref[...] = h1
        g = jax.nn.relu(jnp.dot(h1, wp_ref[...], preferred_element_type=jnp.float32) + bp_ref[...])
        ones = jnp.ones((BR, 1), jnp.float32)
        zeros = jnp.zeros((BR, F - H - 1), jnp.float32)
        g_ref[...] = jnp.concatenate([g, ones, zeros], axis=1)

    return pl.pallas_call(
        body,
        grid=(GRID,),
        in_specs=[
            pl.BlockSpec((BR, H), lambda i: (i, 0)),
            pl.BlockSpec((BR, F), lambda i: (i, 0)),
            pl.BlockSpec((H, H), lambda i: (0, 0)),
            pl.BlockSpec((H, H), lambda i: (0, 0)),
            pl.BlockSpec((1, H), lambda i: (0, 0)),
            pl.BlockSpec((H, H), lambda i: (0, 0)),
            pl.BlockSpec((1, H), lambda i: (0, 0)),
        ],
        out_specs=[
            pl.BlockSpec((BR, H), lambda i: (i, 0)),
            pl.BlockSpec((BR, F), lambda i: (i, 0)),
        ],
        out_shape=[
            jax.ShapeDtypeStruct((NP, H), jnp.float32),
            jax.ShapeDtypeStruct((NP, F), jnp.float32),
        ],
    )(h0, agg, WnhT, WncT, bn1, WpoT, bpo)


def _dense_fin(h1, agg, WnhT, WncT, bno):
    def body(h_ref, a_ref, wh_ref, wc_ref, bn_ref, o_ref):
        i = pl.program_id(0)
        h2 = _node_apply(h_ref[...], a_ref[...], wh_ref[...], wc_ref[...], bn_ref[...])
        rows = i * BR + lax.broadcasted_iota(jnp.int32, (BR, 1), 0)
        h2 = jnp.where(rows < N, h2, 0.0)
        part = jnp.sum(h2, axis=0, keepdims=True)

        @pl.when(i == 0)
        def _():
            o_ref[...] = jnp.zeros((1, H), jnp.float32)

        o_ref[...] += part

        @pl.when(i == GRID - 1)
        def _():
            o_ref[...] = o_ref[...] * (1.0 / N)

    return pl.pallas_call(
        body,
        grid=(GRID,),
        in_specs=[
            pl.BlockSpec((BR, H), lambda i: (i, 0)),
            pl.BlockSpec((BR, F), lambda i: (i, 0)),
            pl.BlockSpec((H, H), lambda i: (0, 0)),
            pl.BlockSpec((H, H), lambda i: (0, 0)),
            pl.BlockSpec((1, H), lambda i: (0, 0)),
        ],
        out_specs=pl.BlockSpec((1, H), lambda i: (0, 0)),
        out_shape=jax.ShapeDtypeStruct((1, H), jnp.float32),
    )(h1, agg, WnhT, WncT, bno)


def _sc_body(gp_hbm, src_hbm, dst_hbm, zer_hbm, out_hbm,
             acc, dstb0, dstb1, srcb0, srcb1, sel_s, sel_d, idx2, gbuf,
             esemd, esems, gsem, ssem):
    c = lax.axis_index("c")
    s = lax.axis_index("s")

    def edge_start(ch, db, sb, i):
        base = s * EPT + ch * CE
        pltpu.async_copy(dst_hbm.at[pl.ds(base, CE)], db, esemd.at[i])
        pltpu.async_copy(src_hbm.at[pl.ds(base, CE)], sb, esems.at[i])

    def edge_wait(db, sb, i):
        pltpu.make_async_copy(dst_hbm.at[pl.ds(0, CE)], db, esemd.at[i]).wait()
        pltpu.make_async_copy(src_hbm.at[pl.ds(0, CE)], sb, esems.at[i]).wait()

    def gather_start(slot, off):
        pltpu.async_copy(gp_hbm.at[sel_s.at[pl.ds(off, GB)]],
                         gbuf.at[slot], gsem.at[slot])

    def gather_wait(slot):
        pltpu.make_async_copy(gp_hbm.at[sel_s.at[pl.ds(0, GB)]],
                              gbuf.at[slot], gsem.at[slot]).wait()

    def scatter_start(slot):
        pltpu.async_copy(gbuf.at[slot], acc.at[idx2.at[slot]], ssem.at[slot],
                         add=True)

    def scatter_wait(slot):
        pltpu.make_async_copy(gbuf.at[slot], acc.at[idx2.at[slot]],
                              ssem.at[slot]).wait()

    def fire(bk, off, k2):
        slot = lax.rem(bk, 3)

        @pl.when(bk >= 3)
        def _():
            scatter_wait(slot)

        for j in range(GB // 16):
            idx2[slot, pl.ds(j * 16, 16)] = sel_d[pl.ds(off + j * 16, 16)]
        gather_start(slot, off)

        @pl.when(k2 >= 2)
        def _():
            pslot = lax.rem(bk - 2, 3)
            gather_wait(pslot)
            scatter_start(pslot)

    for r in range(2):
        lo = (2 * r + c) * CS

        def process_chunk(lo, db, sb, i, carry):
            bk, rem = carry
            edge_wait(db, sb, i)

            def scan_body(sci, cnt):
                parts = []
                for u in range(SCU):
                    o = (sci * SCU + u) * 16
                    d = db[pl.ds(o, 16)]
                    sv = sb[pl.ds(o, 16)]
                    m = (d >= lo) & (d < lo + CS)
                    mi = jnp.where(m, jnp.int32(1), jnp.int32(0))
                    incl = plsc.cumsum(mi)
                    parts.append((d, sv, m, mi, incl))
                for d, sv, m, mi, incl in parts:
                    pos = jnp.where(m, cnt + incl - mi, jnp.int32(SEL - 1))
                    plsc.store_scatter(sel_s, [pos], sv)
                    plsc.store_scatter(sel_d, [pos], d - lo)
                    cnt = cnt + incl[15]
                return cnt

            cnt = lax.fori_loop(0, CE // (16 * SCU), scan_body, rem)
            nbf = cnt // GB

            def batch_body(k2, bk2):
                fire(bk2, k2 * GB, k2)
                return bk2 + 1

            bk = lax.fori_loop(0, nbf, batch_body, bk)

            for dd in (2, 1):
                @pl.when(nbf >= dd)
                def _(dd=dd):
                    pslot = lax.rem(bk - dd, 3)
                    gather_wait(pslot)
                    scatter_start(pslot)

            for j in range(GB // 16):
                sel_s[pl.ds(j * 16, 16)] = sel_s[pl.ds(nbf * GB + j * 16, 16)]
                sel_d[pl.ds(j * 16, 16)] = sel_d[pl.ds(nbf * GB + j * 16, 16)]
            return bk, cnt - nbf * GB

        edge_start(0, dstb0, srcb0, 0)
        pltpu.sync_copy(zer_hbm, acc.at[pl.ds(s * ZR, ZR)])
        plsc.subcore_barrier()

        def pair_body(p, carry):
            ch0 = 2 * p
            edge_start(ch0 + 1, dstb1, srcb1, 1)
            carry = process_chunk(lo, dstb0, srcb0, 0, carry)
            edge_start(ch0 + 2, dstb0, srcb0, 0)
            carry = process_chunk(lo, dstb1, srcb1, 1, carry)
            return carry

        carry = lax.fori_loop(0, (NCH - 1) // 2, pair_body,
                              (jnp.int32(0), jnp.int32(0)))
        bk, rem = process_chunk(lo, dstb0, srcb0, 0, carry)

        z16 = jnp.zeros((16,), jnp.int32)
        d16 = jnp.full((16,), CS, jnp.int32)
        for j in range(GB // 16):
            sel_s[pl.ds(rem + j * 16, 16)] = z16
            sel_d[pl.ds(rem + j * 16, 16)] = d16
        fire(bk, 0, 0)
        gather_wait(lax.rem(bk, 3))
        scatter_start(lax.rem(bk, 3))
        bk = bk + 1

        for jj in range(3):
            @pl.when(bk >= jj + 1)
            def _(jj=jj):
                scatter_wait(lax.rem(bk - 1 - jj, 3))

        plsc.subcore_barrier()
        pltpu.sync_copy(acc.at[pl.ds(s * CR, CR)], out_hbm.at[pl.ds(lo + s * CR, CR)])
        plsc.subcore_barrier()


@functools.cache
def _sc_spmm_call():
    return functools.partial(
        pl.kernel,
        out_type=jax.ShapeDtypeStruct((NP, F), jnp.float32),
        mesh=plsc.VectorSubcoreMesh(core_axis_name="c", subcore_axis_name="s",
                                    num_cores=NSC, num_subcores=NT),
        scratch_types=[
            pltpu.VMEM_SHARED((CS + 16, F), jnp.float32),
            pltpu.VMEM((CE,), jnp.int32),
            pltpu.VMEM((CE,), jnp.int32),
            pltpu.VMEM((CE,), jnp.int32),
            pltpu.VMEM((CE,), jnp.int32),
            pltpu.VMEM((SEL,), jnp.int32),
            pltpu.VMEM((SEL,), jnp.int32),
            pltpu.VMEM((3, GB), jnp.int32),
            pltpu.VMEM((3, GB, F), jnp.float32),
            pltpu.SemaphoreType.DMA((2,)),
            pltpu.SemaphoreType.DMA((2,)),
            pltpu.SemaphoreType.DMA((3,)),
            pltpu.SemaphoreType.DMA((3,)),
        ],
        compiler_params=pltpu.CompilerParams(needs_layout_passes=False),
    )(_sc_body)


def _sc_spmm(gp, src, dst, zer):
    return _sc_spmm_call()(gp, src, dst, zer)


def kernel(nodes_feat, edge_index, edges_feat, nodes_num_norm_sqrt,
           edges_num_norm_sqrt, W_emb, b_emb, Wp1, bp1, Wn1, bn1,
           Wpo, bpo, Wno, bno):
    f32 = jnp.float32
    src = edge_index[0]
    dst = edge_index[1]
    x = jnp.concatenate([nodes_feat, jnp.zeros((NP - N, IN_DIM), f32)], axis=0)
    zer = jnp.zeros((ZR, F), f32)

    h0, g1p = _dense_pre(x, W_emb.T, b_emb[None], Wp1.T, bp1[None])
    agg1 = _sc_spmm(g1p, src, dst, zer)
    h1, g2p = _dense_mid(h0, agg1, Wn1[:, :H].T, Wn1[:, H:].T, bn1[None],
                         Wpo.T, bpo[None])
    agg2 = _sc_spmm(g2p, src, dst, zer)
    return _dense_fin(h1, agg2, Wno[:, :H].T, Wno[:, H:].T, bno[None])


# speedup vs baseline: 1.0718x; 1.0718x over previous
import functools

import jax
import jax.numpy as jnp
from jax import lax
from jax.experimental import pallas as pl
from jax.experimental.pallas import tpu as pltpu
from jax.experimental.pallas import tpu_sc as plsc

N = 50000
E = 800000
IN_DIM = 64
H = 108

BR = 1792
NP = 50176
GRID = NP // BR

NSC = 2
NT = 16
CS = NP // 4
ZR = (CS + 16) // NT
CR = CS // NT
EPT = E // NT
CE = 2000
NCH = EPT // CE
SEL = CE + 176
GB = 64
SCU = 5
F = 128


def _dense_pre(x, WembT, b_emb, Wp1T, bp1):
    def body(x_ref, we_ref, be_ref, wp_ref, bp_ref, h_ref, g_ref):
        h = jnp.dot(x_ref[...], we_ref[...], preferred_element_type=jnp.float32)
        h = h + be_ref[...]
        h_ref[...] = h
        g = jax.nn.relu(jnp.dot(h, wp_ref[...], preferred_element_type=jnp.float32) + bp_ref[...])
        ones = jnp.ones((BR, 1), jnp.float32)
        zeros = jnp.zeros((BR, F - H - 1), jnp.float32)
        g_ref[...] = jnp.concatenate([g, ones, zeros], axis=1)

    return pl.pallas_call(
        body,
        grid=(GRID,),
        in_specs=[
            pl.BlockSpec((BR, IN_DIM), lambda i: (i, 0)),
            pl.BlockSpec((IN_DIM, H), lambda i: (0, 0)),
            pl.BlockSpec((1, H), lambda i: (0, 0)),
            pl.BlockSpec((H, H), lambda i: (0, 0)),
            pl.BlockSpec((1, H), lambda i: (0, 0)),
        ],
        out_specs=[
            pl.BlockSpec((BR, H), lambda i: (i, 0)),
            pl.BlockSpec((BR, F), lambda i: (i, 0)),
        ],
        out_shape=[
            jax.ShapeDtypeStruct((NP, H), jnp.float32),
            jax.ShapeDtypeStruct((NP, F), jnp.float32),
        ],
    )(x, WembT, b_emb, Wp1T, bp1)


def _node_apply(h, agg, wh, wc, bn):
    cnt = jnp.maximum(agg[:, H:H + 1], 1.0)
    c = agg[:, :H] / cnt
    bundle = (jnp.dot(h, wh, preferred_element_type=jnp.float32)
              + jnp.dot(c, wc, preferred_element_type=jnp.float32) + bn)
    nrm = jnp.maximum(jnp.sqrt(jnp.sum(bundle * bundle, axis=1, keepdims=True)), 1e-12)
    return h + jax.nn.relu(bundle / nrm)


def _dense_mid(h0, agg, WnhT, WncT, bn1, WpoT, bpo):
    def body(h_ref, a_ref, wh_ref, wc_ref, bn_ref, wp_ref, bp_ref, h1_ref, g_ref):
        h1 = _node_apply(h_ref[...], a_ref[...], wh_ref[...], wc_ref[...], bn_ref[...])
        h1_ref[...] = h1
        g = jax.nn.relu(jnp.dot(h1, wp_ref[...], preferred_element_type=jnp.float32) + bp_ref[...])
        ones = jnp.ones((BR, 1), jnp.float32)
        zeros = jnp.zeros((BR, F - H - 1), jnp.float32)
        g_ref[...] = jnp.concatenate([g, ones, zeros], axis=1)

    return pl.pallas_call(
        body,
        grid=(GRID,),
        in_specs=[
            pl.BlockSpec((BR, H), lambda i: (i, 0)),
            pl.BlockSpec((BR, F), lambda i: (i, 0)),
            pl.BlockSpec((H, H), lambda i: (0, 0)),
            pl.BlockSpec((H, H), lambda i: (0, 0)),
            pl.BlockSpec((1, H), lambda i: (0, 0)),
            pl.BlockSpec((H, H), lambda i: (0, 0)),
            pl.BlockSpec((1, H), lambda i: (0, 0)),
        ],
        out_specs=[
            pl.BlockSpec((BR, H), lambda i: (i, 0)),
            pl.BlockSpec((BR, F), lambda i: (i, 0)),
        ],
        out_shape=[
            jax.ShapeDtypeStruct((NP, H), jnp.float32),
            jax.ShapeDtypeStruct((NP, F), jnp.float32),
        ],
    )(h0, agg, WnhT, WncT, bn1, WpoT, bpo)


def _dense_fin(h1, agg, WnhT, WncT, bno):
    def body(h_ref, a_ref, wh_ref, wc_ref, bn_ref, o_ref):
        i = pl.program_id(0)
        h2 = _node_apply(h_ref[...], a_ref[...], wh_ref[...], wc_ref[...], bn_ref[...])
        rows = i * BR + lax.broadcasted_iota(jnp.int32, (BR, 1), 0)
        h2 = jnp.where(rows < N, h2, 0.0)
        part = jnp.sum(h2, axis=0, keepdims=True)

        @pl.when(i == 0)
        def _():
            o_ref[...] = jnp.zeros((1, H), jnp.float32)

        o_ref[...] += part

        @pl.when(i == GRID - 1)
        def _():
            o_ref[...] = o_ref[...] * (1.0 / N)

    return pl.pallas_call(
        body,
        grid=(GRID,),
        in_specs=[
            pl.BlockSpec((BR, H), lambda i: (i, 0)),
            pl.BlockSpec((BR, F), lambda i: (i, 0)),
            pl.BlockSpec((H, H), lambda i: (0, 0)),
            pl.BlockSpec((H, H), lambda i: (0, 0)),
            pl.BlockSpec((1, H), lambda i: (0, 0)),
        ],
        out_specs=pl.BlockSpec((1, H), lambda i: (0, 0)),
        out_shape=jax.ShapeDtypeStruct((1, H), jnp.float32),
    )(h1, agg, WnhT, WncT, bno)


def _sc_body(gp_hbm, src_hbm, dst_hbm, zer_hbm, out_hbm,
             acc, dstb0, dstb1, srcb0, srcb1, sel_s, sel_d, idx2, gbuf,
             esemd, esems, gsem, ssem):
    c = lax.axis_index("c")
    s = lax.axis_index("s")

    def edge_start(ch, db, sb, i):
        base = s * EPT + ch * CE
        pltpu.async_copy(dst_hbm.at[pl.ds(base, CE)], db, esemd.at[i])
        pltpu.async_copy(src_hbm.at[pl.ds(base, CE)], sb, esems.at[i])

    def edge_wait(db, sb, i):
        pltpu.make_async_copy(dst_hbm.at[pl.ds(0, CE)], db, esemd.at[i]).wait()
        pltpu.make_async_copy(src_hbm.at[pl.ds(0, CE)], sb, esems.at[i]).wait()

    def gather_start(slot, off):
        pltpu.async_copy(gp_hbm.at[sel_s.at[pl.ds(off, GB)]],
                         gbuf.at[slot], gsem.at[slot])

    def gather_wait(slot):
        pltpu.make_async_copy(gp_hbm.at[sel_s.at[pl.ds(0, GB)]],
                              gbuf.at[slot], gsem.at[slot]).wait()

    def scatter_start(slot):
        pltpu.async_copy(gbuf.at[slot], acc.at[idx2.at[slot]], ssem.at[slot],
                         add=True)

    def scatter_wait(slot):
        pltpu.make_async_copy(gbuf.at[slot], acc.at[idx2.at[slot]],
                              ssem.at[slot]).wait()

    def fire(bk, off, k2):
        slot = bk & 1

        @pl.when(bk >= 2)
        def _():
            scatter_wait(slot)

        for j in range(GB // 16):
            idx2[slot, pl.ds(j * 16, 16)] = sel_d[pl.ds(off + j * 16, 16)]
        gather_start(slot, off)

        @pl.when(k2 >= 1)
        def _():
            pslot = (bk - 1) & 1
            gather_wait(pslot)
            scatter_start(pslot)

    for r in range(2):
        lo = (2 * r + c) * CS

        def process_chunk(lo, db, sb, i, carry):
            bk, rem = carry
            edge_wait(db, sb, i)

            def scan_body(sci, cnt):
                parts = []
                for u in range(SCU):
                    o = (sci * SCU + u) * 16
                    d = db[pl.ds(o, 16)]
                    sv = sb[pl.ds(o, 16)]
                    m = (d >= lo) & (d < lo + CS)
                    mi = jnp.where(m, jnp.int32(1), jnp.int32(0))
                    incl = plsc.cumsum(mi)
                    parts.append((d, sv, m, mi, incl))
                for d, sv, m, mi, incl in parts:
                    pos = jnp.where(m, cnt + incl - mi, jnp.int32(SEL - 1))
                    plsc.store_scatter(sel_s, [pos], sv)
                    plsc.store_scatter(sel_d, [pos], d - lo)
                    cnt = cnt + incl[15]
                return cnt

            cnt = lax.fori_loop(0, CE // (16 * SCU), scan_body, rem)
            nbf = cnt // GB

            def batch_body(k2, bk2):
                fire(bk2, k2 * GB, k2)
                return bk2 + 1

            bk = lax.fori_loop(0, nbf, batch_body, bk)

            @pl.when(nbf >= 1)
            def _():
                pslot = (bk - 1) & 1
                gather_wait(pslot)
                scatter_start(pslot)

            for j in range(GB // 16):
                sel_s[pl.ds(j * 16, 16)] = sel_s[pl.ds(nbf * GB + j * 16, 16)]
                sel_d[pl.ds(j * 16, 16)] = sel_d[pl.ds(nbf * GB + j * 16, 16)]
            return bk, cnt - nbf * GB

        edge_start(0, dstb0, srcb0, 0)
        pltpu.sync_copy(zer_hbm, acc.at[pl.ds(s * ZR, ZR)])
        plsc.subcore_barrier()

        def pair_body(p, carry):
            ch0 = 2 * p
            edge_start(ch0 + 1, dstb1, srcb1, 1)
            carry = process_chunk(lo, dstb0, srcb0, 0, carry)
            edge_start(ch0 + 2, dstb0, srcb0, 0)
            carry = process_chunk(lo, dstb1, srcb1, 1, carry)
            return carry

        carry = lax.fori_loop(0, (NCH - 1) // 2, pair_body,
                              (jnp.int32(0), jnp.int32(0)))
        bk, rem = process_chunk(lo, dstb0, srcb0, 0, carry)

        z16 = jnp.zeros((16,), jnp.int32)
        d16 = jnp.full((16,), CS, jnp.int32)
        for j in range(GB // 16):
            sel_s[pl.ds(rem + j * 16, 16)] = z16
            sel_d[pl.ds(rem + j * 16, 16)] = d16
        fire(bk, 0, 0)
        gather_wait(bk & 1)
        scatter_start(bk & 1)
        bk = bk + 1

        for jj in range(2):
            @pl.when(bk >= jj + 1)
            def _(jj=jj):
                scatter_wait((bk - 1 - jj) & 1)

        plsc.subcore_barrier()
        pltpu.sync_copy(acc.at[pl.ds(s * CR, CR)], out_hbm.at[pl.ds(lo + s * CR, CR)])
        plsc.subcore_barrier()


@functools.cache
def _sc_spmm_call():
    return functools.partial(
        pl.kernel,
        out_type=jax.ShapeDtypeStruct((NP, F), jnp.float32),
        mesh=plsc.VectorSubcoreMesh(core_axis_name="c", subcore_axis_name="s",
                                    num_cores=NSC, num_subcores=NT),
        scratch_types=[
            pltpu.VMEM_SHARED((CS + 16, F), jnp.float32),
            pltpu.VMEM((CE,), jnp.int32),
            pltpu.VMEM((CE,), jnp.int32),
            pltpu.VMEM((CE,), jnp.int32),
            pltpu.VMEM((CE,), jnp.int32),
            pltpu.VMEM((SEL,), jnp.int32),
            pltpu.VMEM((SEL,), jnp.int32),
            pltpu.VMEM((2, GB), jnp.int32),
            pltpu.VMEM((2, GB, F), jnp.float32),
            pltpu.SemaphoreType.DMA((2,)),
            pltpu.SemaphoreType.DMA((2,)),
            pltpu.SemaphoreType.DMA((2,)),
            pltpu.SemaphoreType.DMA((2,)),
        ],
        compiler_params=pltpu.CompilerParams(needs_layout_passes=False),
    )(_sc_body)


def _sc_spmm(gp, src, dst, zer):
    return _sc_spmm_call()(gp, src, dst, zer)


def kernel(nodes_feat, edge_index, edges_feat, nodes_num_norm_sqrt,
           edges_num_norm_sqrt, W_emb, b_emb, Wp1, bp1, Wn1, bn1,
           Wpo, bpo, Wno, bno):
    f32 = jnp.float32
    src = edge_index[0]
    dst = edge_index[1]
    x = jnp.concatenate([nodes_feat, jnp.zeros((NP - N, IN_DIM), f32)], axis=0)
    zer = jnp.zeros((ZR, F), f32)

    h0, g1p = _dense_pre(x, W_emb.T, b_emb[None], Wp1.T, bp1[None])
    agg1 = _sc_spmm(g1p, src, dst, zer)
    h1, g2p = _dense_mid(h0, agg1, Wn1[:, :H].T, Wn1[:, H:].T, bn1[None],
                         Wpo.T, bpo[None])
    agg2 = _sc_spmm(g2p, src, dst, zer)
    return _dense_fin(h1, agg2, Wno[:, :H].T, Wno[:, H:].T, bno[None])


# speedup vs baseline: 1.1015x; 1.0277x over previous
import functools

import jax
import jax.numpy as jnp
from jax import lax
from jax.experimental import pallas as pl
from jax.experimental.pallas import tpu as pltpu
from jax.experimental.pallas import tpu_sc as plsc

N = 50000
E = 800000
IN_DIM = 64
H = 108

BR = 3584
NP = 50176
GRID = NP // BR

NSC = 2
NT = 16
CS = NP // 4
ZR = (CS + 16) // NT
CR = CS // NT
EPT = E // NT
CE = 2000
NCH = EPT // CE
SEL = CE + 176
GB = 64
SCU = 5
F = 128


def _dense_pre(x, WembT, b_emb, Wp1T, bp1):
    def body(x_ref, we_ref, be_ref, wp_ref, bp_ref, h_ref, g_ref):
        h = jnp.dot(x_ref[...], we_ref[...], preferred_element_type=jnp.float32)
        h = h + be_ref[...]
        h_ref[...] = h
        g = jax.nn.relu(jnp.dot(h, wp_ref[...], preferred_element_type=jnp.float32) + bp_ref[...])
        ones = jnp.ones((BR, 1), jnp.float32)
        zeros = jnp.zeros((BR, F - H - 1), jnp.float32)
        g_ref[...] = jnp.concatenate([g, ones, zeros], axis=1)

    return pl.pallas_call(
        body,
        grid=(GRID,),
        in_specs=[
            pl.BlockSpec((BR, IN_DIM), lambda i: (i, 0)),
            pl.BlockSpec((IN_DIM, H), lambda i: (0, 0)),
            pl.BlockSpec((1, H), lambda i: (0, 0)),
            pl.BlockSpec((H, H), lambda i: (0, 0)),
            pl.BlockSpec((1, H), lambda i: (0, 0)),
        ],
        out_specs=[
            pl.BlockSpec((BR, H), lambda i: (i, 0)),
            pl.BlockSpec((BR, F), lambda i: (i, 0)),
        ],
        out_shape=[
            jax.ShapeDtypeStruct((NP, H), jnp.float32),
            jax.ShapeDtypeStruct((NP, F), jnp.float32),
        ],
    )(x, WembT, b_emb, Wp1T, bp1)


def _node_apply(h, agg, wh, wc, bn):
    cnt = jnp.maximum(agg[:, H:H + 1], 1.0)
    c = agg[:, :H] / cnt
    bundle = (jnp.dot(h, wh, preferred_element_type=jnp.float32)
              + jnp.dot(c, wc, preferred_element_type=jnp.float32) + bn)
    nrm = jnp.maximum(jnp.sqrt(jnp.sum(bundle * bundle, axis=1, keepdims=True)), 1e-12)
    return h + jax.nn.relu(bundle / nrm)


def _dense_mid(h0, agg, WnhT, WncT, bn1, WpoT, bpo):
    def body(h_ref, a_ref, wh_ref, wc_ref, bn_ref, wp_ref, bp_ref, h1_ref, g_ref):
        h1 = _node_apply(h_ref[...], a_ref[...], wh_ref[...], wc_ref[...], bn_ref[...])
        h1_ref[...] = h1
        g = jax.nn.relu(jnp.dot(h1, wp_ref[...], preferred_element_type=jnp.float32) + bp_ref[...])
        ones = jnp.ones((BR, 1), jnp.float32)
        zeros = jnp.zeros((BR, F - H - 1), jnp.float32)
        g_ref[...] = jnp.concatenate([g, ones, zeros], axis=1)

    return pl.pallas_call(
        body,
        grid=(GRID,),
        in_specs=[
            pl.BlockSpec((BR, H), lambda i: (i, 0)),
            pl.BlockSpec((BR, F), lambda i: (i, 0)),
            pl.BlockSpec((H, H), lambda i: (0, 0)),
            pl.BlockSpec((H, H), lambda i: (0, 0)),
            pl.BlockSpec((1, H), lambda i: (0, 0)),
            pl.BlockSpec((H, H), lambda i: (0, 0)),
            pl.BlockSpec((1, H), lambda i: (0, 0)),
        ],
        out_specs=[
            pl.BlockSpec((BR, H), lambda i: (i, 0)),
            pl.BlockSpec((BR, F), lambda i: (i, 0)),
        ],
        out_shape=[
            jax.ShapeDtypeStruct((NP, H), jnp.float32),
            jax.ShapeDtypeStruct((NP, F), jnp.float32),
        ],
    )(h0, agg, WnhT, WncT, bn1, WpoT, bpo)


def _dense_fin(h1, agg, WnhT, WncT, bno):
    def body(h_ref, a_ref, wh_ref, wc_ref, bn_ref, o_ref):
        i = pl.program_id(0)
        h2 = _node_apply(h_ref[...], a_ref[...], wh_ref[...], wc_ref[...], bn_ref[...])
        rows = i * BR + lax.broadcasted_iota(jnp.int32, (BR, 1), 0)
        h2 = jnp.where(rows < N, h2, 0.0)
        part = jnp.sum(h2, axis=0, keepdims=True)

        @pl.when(i == 0)
        def _():
            o_ref[...] = jnp.zeros((1, H), jnp.float32)

        o_ref[...] += part

        @pl.when(i == GRID - 1)
        def _():
            o_ref[...] = o_ref[...] * (1.0 / N)

    return pl.pallas_call(
        body,
        grid=(GRID,),
        in_specs=[
            pl.BlockSpec((BR, H), lambda i: (i, 0)),
            pl.BlockSpec((BR, F), lambda i: (i, 0)),
            pl.BlockSpec((H, H), lambda i: (0, 0)),
            pl.BlockSpec((H, H), lambda i: (0, 0)),
            pl.BlockSpec((1, H), lambda i: (0, 0)),
        ],
        out_specs=pl.BlockSpec((1, H), lambda i: (0, 0)),
        out_shape=jax.ShapeDtypeStruct((1, H), jnp.float32),
    )(h1, agg, WnhT, WncT, bno)


def _sc_body(gp_hbm, src_hbm, dst_hbm, zer_hbm, out_hbm,
             acc, dstb0, dstb1, srcb0, srcb1, sel_s, sel_d, idx2, gbuf,
             esemd, esems, gsem, ssem):
    c = lax.axis_index("c")
    s = lax.axis_index("s")

    def edge_start(ch, db, sb, i):
        base = s * EPT + ch * CE
        pltpu.async_copy(dst_hbm.at[pl.ds(base, CE)], db, esemd.at[i])
        pltpu.async_copy(src_hbm.at[pl.ds(base, CE)], sb, esems.at[i])

    def edge_wait(db, sb, i):
        pltpu.make_async_copy(dst_hbm.at[pl.ds(0, CE)], db, esemd.at[i]).wait()
        pltpu.make_async_copy(src_hbm.at[pl.ds(0, CE)], sb, esems.at[i]).wait()

    def gather_start(slot, off):
        pltpu.async_copy(gp_hbm.at[sel_s.at[pl.ds(off, GB)]],
                         gbuf.at[slot], gsem.at[slot])

    def gather_wait(slot):
        pltpu.make_async_copy(gp_hbm.at[sel_s.at[pl.ds(0, GB)]],
                              gbuf.at[slot], gsem.at[slot]).wait()

    def scatter_start(slot):
        pltpu.async_copy(gbuf.at[slot], acc.at[idx2.at[slot]], ssem.at[slot],
                         add=True)

    def scatter_wait(slot):
        pltpu.make_async_copy(gbuf.at[slot], acc.at[idx2.at[slot]],
                              ssem.at[slot]).wait()

    def fire(bk, off, k2):
        slot = bk & 1

        @pl.when(bk >= 2)
        def _():
            scatter_wait(slot)

        for j in range(GB // 16):
            idx2[slot, pl.ds(j * 16, 16)] = sel_d[pl.ds(off + j * 16, 16)]
        gather_start(slot, off)

        @pl.when(k2 >= 1)
        def _():
            pslot = (bk - 1) & 1
            gather_wait(pslot)
            scatter_start(pslot)

    for r in range(2):
        lo = (2 * r + c) * CS

        def process_chunk(lo, db, sb, i, carry):
            bk, rem = carry
            edge_wait(db, sb, i)

            def scan_body(sci, cnt):
                parts = []
                for u in range(SCU):
                    o = (sci * SCU + u) * 16
                    d = db[pl.ds(o, 16)]
                    sv = sb[pl.ds(o, 16)]
                    m = (d >= lo) & (d < lo + CS)
                    mi = jnp.where(m, jnp.int32(1), jnp.int32(0))
                    incl = plsc.cumsum(mi)
                    parts.append((d, sv, m, mi, incl))
                for d, sv, m, mi, incl in parts:
                    pos = jnp.where(m, cnt + incl - mi, jnp.int32(SEL - 1))
                    plsc.store_scatter(sel_s, [pos], sv)
                    plsc.store_scatter(sel_d, [pos], d - lo)
                    cnt = cnt + incl[15]
                return cnt

            cnt = lax.fori_loop(0, CE // (16 * SCU), scan_body, rem)
            nbf = cnt // GB

            def batch_body(k2, bk2):
                fire(bk2, k2 * GB, k2)
                return bk2 + 1

            bk = lax.fori_loop(0, nbf, batch_body, bk)

            @pl.when(nbf >= 1)
            def _():
                pslot = (bk - 1) & 1
                gather_wait(pslot)
                scatter_start(pslot)

            for j in range(GB // 16):
                sel_s[pl.ds(j * 16, 16)] = sel_s[pl.ds(nbf * GB + j * 16, 16)]
                sel_d[pl.ds(j * 16, 16)] = sel_d[pl.ds(nbf * GB + j * 16, 16)]
            return bk, cnt - nbf * GB

        edge_start(0, dstb0, srcb0, 0)
        pltpu.sync_copy(zer_hbm, acc.at[pl.ds(s * ZR, ZR)])
        plsc.subcore_barrier()

        def pair_body(p, carry):
            ch0 = 2 * p
            edge_start(ch0 + 1, dstb1, srcb1, 1)
            carry = process_chunk(lo, dstb0, srcb0, 0, carry)
            edge_start(ch0 + 2, dstb0, srcb0, 0)
            carry = process_chunk(lo, dstb1, srcb1, 1, carry)
            return carry

        carry = lax.fori_loop(0, (NCH - 1) // 2, pair_body,
                              (jnp.int32(0), jnp.int32(0)))
        bk, rem = process_chunk(lo, dstb0, srcb0, 0, carry)

        z16 = jnp.zeros((16,), jnp.int32)
        d16 = jnp.full((16,), CS, jnp.int32)
        for j in range(GB // 16):
            sel_s[pl.ds(rem + j * 16, 16)] = z16
            sel_d[pl.ds(rem + j * 16, 16)] = d16
        fire(bk, 0, 0)
        gather_wait(bk & 1)
        scatter_start(bk & 1)
        bk = bk + 1

        for jj in range(2):
            @pl.when(bk >= jj + 1)
            def _(jj=jj):
                scatter_wait((bk - 1 - jj) & 1)

        plsc.subcore_barrier()
        pltpu.sync_copy(acc.at[pl.ds(s * CR, CR)], out_hbm.at[pl.ds(lo + s * CR, CR)])
        plsc.subcore_barrier()


@functools.cache
def _sc_spmm_call():
    return functools.partial(
        pl.kernel,
        out_type=jax.ShapeDtypeStruct((NP, F), jnp.float32),
        mesh=plsc.VectorSubcoreMesh(core_axis_name="c", subcore_axis_name="s",
                                    num_cores=NSC, num_subcores=NT),
        scratch_types=[
            pltpu.VMEM_SHARED((CS + 16, F), jnp.float32),
            pltpu.VMEM((CE,), jnp.int32),
            pltpu.VMEM((CE,), jnp.int32),
            pltpu.VMEM((CE,), jnp.int32),
            pltpu.VMEM((CE,), jnp.int32),
            pltpu.VMEM((SEL,), jnp.int32),
            pltpu.VMEM((SEL,), jnp.int32),
            pltpu.VMEM((2, GB), jnp.int32),
            pltpu.VMEM((2, GB, F), jnp.float32),
            pltpu.SemaphoreType.DMA((2,)),
            pltpu.SemaphoreType.DMA((2,)),
            pltpu.SemaphoreType.DMA((2,)),
            pltpu.SemaphoreType.DMA((2,)),
        ],
        compiler_params=pltpu.CompilerParams(needs_layout_passes=False),
    )(_sc_body)


def _sc_spmm(gp, src, dst, zer):
    return _sc_spmm_call()(gp, src, dst, zer)


def kernel(nodes_feat, edge_index, edges_feat, nodes_num_norm_sqrt,
           edges_num_norm_sqrt, W_emb, b_emb, Wp1, bp1, Wn1, bn1,
           Wpo, bpo, Wno, bno):
    f32 = jnp.float32
    src = edge_index[0]
    dst = edge_index[1]
    x = jnp.concatenate([nodes_feat, jnp.zeros((NP - N, IN_DIM), f32)], axis=0)
    zer = jnp.zeros((ZR, F), f32)

    h0, g1p = _dense_pre(x, W_emb.T, b_emb[None], Wp1.T, bp1[None])
    agg1 = _sc_spmm(g1p, src, dst, zer)
    h1, g2p = _dense_mid(h0, agg1, Wn1[:, :H].T, Wn1[:, H:].T, bn1[None],
                         Wpo.T, bpo[None])
    agg2 = _sc_spmm(g2p, src, dst, zer)
    return _dense_fin(h1, agg2, Wno[:, :H].T, Wno[:, H:].T, bno[None])


# speedup vs baseline: 1.1065x; 1.0046x over previous
import functools

import jax
import jax.numpy as jnp
from jax import lax
from jax.experimental import pallas as pl
from jax.experimental.pallas import tpu as pltpu
from jax.experimental.pallas import tpu_sc as plsc

N = 50000
E = 800000
IN_DIM = 64
H = 108

BR = 7168
NP = 50176
GRID = NP // BR

NSC = 2
NT = 16
CS = NP // 4
ZR = (CS + 16) // NT
CR = CS // NT
EPT = E // NT
CE = 2000
NCH = EPT // CE
SEL = CE + 176
GB = 64
SCU = 5
F = 128


def _dense_pre(x, WembT, b_emb, Wp1T, bp1):
    def body(x_ref, we_ref, be_ref, wp_ref, bp_ref, h_ref, g_ref):
        h = jnp.dot(x_ref[...], we_ref[...], preferred_element_type=jnp.float32)
        h = h + be_ref[...]
        h_ref[...] = h
        g = jax.nn.relu(jnp.dot(h, wp_ref[...], preferred_element_type=jnp.float32) + bp_ref[...])
        ones = jnp.ones((BR, 1), jnp.float32)
        zeros = jnp.zeros((BR, F - H - 1), jnp.float32)
        g_ref[...] = jnp.concatenate([g, ones, zeros], axis=1)

    return pl.pallas_call(
        body,
        grid=(GRID,),
        in_specs=[
            pl.BlockSpec((BR, IN_DIM), lambda i: (i, 0)),
            pl.BlockSpec((IN_DIM, H), lambda i: (0, 0)),
            pl.BlockSpec((1, H), lambda i: (0, 0)),
            pl.BlockSpec((H, H), lambda i: (0, 0)),
            pl.BlockSpec((1, H), lambda i: (0, 0)),
        ],
        out_specs=[
            pl.BlockSpec((BR, H), lambda i: (i, 0)),
            pl.BlockSpec((BR, F), lambda i: (i, 0)),
        ],
        out_shape=[
            jax.ShapeDtypeStruct((NP, H), jnp.float32),
            jax.ShapeDtypeStruct((NP, F), jnp.float32),
        ],
    )(x, WembT, b_emb, Wp1T, bp1)


def _node_apply(h, agg, wh, wc, bn):
    cnt = jnp.maximum(agg[:, H:H + 1], 1.0)
    c = agg[:, :H] / cnt
    bundle = (jnp.dot(h, wh, preferred_element_type=jnp.float32)
              + jnp.dot(c, wc, preferred_element_type=jnp.float32) + bn)
    nrm = jnp.maximum(jnp.sqrt(jnp.sum(bundle * bundle, axis=1, keepdims=True)), 1e-12)
    return h + jax.nn.relu(bundle / nrm)


def _dense_mid(h0, agg, WnhT, WncT, bn1, WpoT, bpo):
    def body(h_ref, a_ref, wh_ref, wc_ref, bn_ref, wp_ref, bp_ref, h1_ref, g_ref):
        h1 = _node_apply(h_ref[...], a_ref[...], wh_ref[...], wc_ref[...], bn_ref[...])
        h1_ref[...] = h1
        g = jax.nn.relu(jnp.dot(h1, wp_ref[...], preferred_element_type=jnp.float32) + bp_ref[...])
        ones = jnp.ones((BR, 1), jnp.float32)
        zeros = jnp.zeros((BR, F - H - 1), jnp.float32)
        g_ref[...] = jnp.concatenate([g, ones, zeros], axis=1)

    return pl.pallas_call(
        body,
        grid=(GRID,),
        in_specs=[
            pl.BlockSpec((BR, H), lambda i: (i, 0)),
            pl.BlockSpec((BR, F), lambda i: (i, 0)),
            pl.BlockSpec((H, H), lambda i: (0, 0)),
            pl.BlockSpec((H, H), lambda i: (0, 0)),
            pl.BlockSpec((1, H), lambda i: (0, 0)),
            pl.BlockSpec((H, H), lambda i: (0, 0)),
            pl.BlockSpec((1, H), lambda i: (0, 0)),
        ],
        out_specs=[
            pl.BlockSpec((BR, H), lambda i: (i, 0)),
            pl.BlockSpec((BR, F), lambda i: (i, 0)),
        ],
        out_shape=[
            jax.ShapeDtypeStruct((NP, H), jnp.float32),
            jax.ShapeDtypeStruct((NP, F), jnp.float32),
        ],
    )(h0, agg, WnhT, WncT, bn1, WpoT, bpo)


def _dense_fin(h1, agg, WnhT, WncT, bno):
    def body(h_ref, a_ref, wh_ref, wc_ref, bn_ref, o_ref):
        i = pl.program_id(0)
        h2 = _node_apply(h_ref[...], a_ref[...], wh_ref[...], wc_ref[...], bn_ref[...])
        rows = i * BR + lax.broadcasted_iota(jnp.int32, (BR, 1), 0)
        h2 = jnp.where(rows < N, h2, 0.0)
        part = jnp.sum(h2, axis=0, keepdims=True)

        @pl.when(i == 0)
        def _():
            o_ref[...] = jnp.zeros((1, H), jnp.float32)

        o_ref[...] += part

        @pl.when(i == GRID - 1)
        def _():
            o_ref[...] = o_ref[...] * (1.0 / N)

    return pl.pallas_call(
        body,
        grid=(GRID,),
        in_specs=[
            pl.BlockSpec((BR, H), lambda i: (i, 0)),
            pl.BlockSpec((BR, F), lambda i: (i, 0)),
            pl.BlockSpec((H, H), lambda i: (0, 0)),
            pl.BlockSpec((H, H), lambda i: (0, 0)),
            pl.BlockSpec((1, H), lambda i: (0, 0)),
        ],
        out_specs=pl.BlockSpec((1, H), lambda i: (0, 0)),
        out_shape=jax.ShapeDtypeStruct((1, H), jnp.float32),
    )(h1, agg, WnhT, WncT, bno)


def _sc_body(gp_hbm, src_hbm, dst_hbm, zer_hbm, out_hbm,
             acc, dstb0, dstb1, srcb0, srcb1, sel_s, sel_d, idx2, gbuf,
             esemd, esems, gsem, ssem):
    c = lax.axis_index("c")
    s = lax.axis_index("s")

    def edge_start(ch, db, sb, i):
        base = s * EPT + ch * CE
        pltpu.async_copy(dst_hbm.at[pl.ds(base, CE)], db, esemd.at[i])
        pltpu.async_copy(src_hbm.at[pl.ds(base, CE)], sb, esems.at[i])

    def edge_wait(db, sb, i):
        pltpu.make_async_copy(dst_hbm.at[pl.ds(0, CE)], db, esemd.at[i]).wait()
        pltpu.make_async_copy(src_hbm.at[pl.ds(0, CE)], sb, esems.at[i]).wait()

    def gather_start(slot, off):
        pltpu.async_copy(gp_hbm.at[sel_s.at[pl.ds(off, GB)]],
                         gbuf.at[slot], gsem.at[slot])

    def gather_wait(slot):
        pltpu.make_async_copy(gp_hbm.at[sel_s.at[pl.ds(0, GB)]],
                              gbuf.at[slot], gsem.at[slot]).wait()

    def scatter_start(slot):
        pltpu.async_copy(gbuf.at[slot], acc.at[idx2.at[slot]], ssem.at[slot],
                         add=True)

    def scatter_wait(slot):
        pltpu.make_async_copy(gbuf.at[slot], acc.at[idx2.at[slot]],
                              ssem.at[slot]).wait()

    def fire(bk, off, k2):
        slot = bk & 1

        @pl.when(bk >= 2)
        def _():
            scatter_wait(slot)

        for j in range(GB // 16):
            idx2[slot, pl.ds(j * 16, 16)] = sel_d[pl.ds(off + j * 16, 16)]
        gather_start(slot, off)

        @pl.when(k2 >= 1)
        def _():
            pslot = (bk - 1) & 1
            gather_wait(pslot)
            scatter_start(pslot)

    for r in range(2):
        lo = (2 * r + c) * CS

        def process_chunk(lo, db, sb, i, carry):
            bk, rem = carry
            edge_wait(db, sb, i)

            def scan_body(sci, cnt):
                parts = []
                for u in range(SCU):
                    o = (sci * SCU + u) * 16
                    d = db[pl.ds(o, 16)]
                    sv = sb[pl.ds(o, 16)]
                    m = (d >= lo) & (d < lo + CS)
                    mi = jnp.where(m, jnp.int32(1), jnp.int32(0))
                    incl = plsc.cumsum(mi)
                    parts.append((d, sv, m, mi, incl))
                for d, sv, m, mi, incl in parts:
                    pos = jnp.where(m, cnt + incl - mi, jnp.int32(SEL - 1))
                    plsc.store_scatter(sel_s, [pos], sv)
                    plsc.store_scatter(sel_d, [pos], d - lo)
                    cnt = cnt + incl[15]
                return cnt

            cnt = lax.fori_loop(0, CE // (16 * SCU), scan_body, rem)
            nbf = cnt // GB

            def batch_body(k2, bk2):
                fire(bk2, k2 * GB, k2)
                return bk2 + 1

            bk = lax.fori_loop(0, nbf, batch_body, bk)

            @pl.when(nbf >= 1)
            def _():
                pslot = (bk - 1) & 1
                gather_wait(pslot)
                scatter_start(pslot)

            for j in range(GB // 16):
                sel_s[pl.ds(j * 16, 16)] = sel_s[pl.ds(nbf * GB + j * 16, 16)]
                sel_d[pl.ds(j * 16, 16)] = sel_d[pl.ds(nbf * GB + j * 16, 16)]
            return bk, cnt - nbf * GB

        edge_start(0, dstb0, srcb0, 0)
        pltpu.sync_copy(zer_hbm, acc.at[pl.ds(s * ZR, ZR)])
        plsc.subcore_barrier()

        def pair_body(p, carry):
            ch0 = 2 * p
            edge_start(ch0 + 1, dstb1, srcb1, 1)
            carry = process_chunk(lo, dstb0, srcb0, 0, carry)
            edge_start(ch0 + 2, dstb0, srcb0, 0)
            carry = process_chunk(lo, dstb1, srcb1, 1, carry)
            return carry

        carry = lax.fori_loop(0, (NCH - 1) // 2, pair_body,
                              (jnp.int32(0), jnp.int32(0)))
        bk, rem = process_chunk(lo, dstb0, srcb0, 0, carry)

        z16 = jnp.zeros((16,), jnp.int32)
        d16 = jnp.full((16,), CS, jnp.int32)
        for j in range(GB // 16):
            sel_s[pl.ds(rem + j * 16, 16)] = z16
            sel_d[pl.ds(rem + j * 16, 16)] = d16
        fire(bk, 0, 0)
        gather_wait(bk & 1)
        scatter_start(bk & 1)
        bk = bk + 1

        for jj in range(2):
            @pl.when(bk >= jj + 1)
            def _(jj=jj):
                scatter_wait((bk - 1 - jj) & 1)

        plsc.subcore_barrier()
        pltpu.sync_copy(acc.at[pl.ds(s * CR, CR)], out_hbm.at[pl.ds(lo + s * CR, CR)])
        plsc.subcore_barrier()


@functools.cache
def _sc_spmm_call():
    return functools.partial(
        pl.kernel,
        out_type=jax.ShapeDtypeStruct((NP, F), jnp.float32),
        mesh=plsc.VectorSubcoreMesh(core_axis_name="c", subcore_axis_name="s",
                                    num_cores=NSC, num_subcores=NT),
        scratch_types=[
            pltpu.VMEM_SHARED((CS + 16, F), jnp.float32),
            pltpu.VMEM((CE,), jnp.int32),
            pltpu.VMEM((CE,), jnp.int32),
            pltpu.VMEM((CE,), jnp.int32),
            pltpu.VMEM((CE,), jnp.int32),
            pltpu.VMEM((SEL,), jnp.int32),
            pltpu.VMEM((SEL,), jnp.int32),
            pltpu.VMEM((2, GB), jnp.int32),
            pltpu.VMEM((2, GB, F), jnp.float32),
            pltpu.SemaphoreType.DMA((2,)),
            pltpu.SemaphoreType.DMA((2,)),
            pltpu.SemaphoreType.DMA((2,)),
            pltpu.SemaphoreType.DMA((2,)),
        ],
        compiler_params=pltpu.CompilerParams(needs_layout_passes=False),
    )(_sc_body)


def _sc_spmm(gp, src, dst, zer):
    return _sc_spmm_call()(gp, src, dst, zer)


def kernel(nodes_feat, edge_index, edges_feat, nodes_num_norm_sqrt,
           edges_num_norm_sqrt, W_emb, b_emb, Wp1, bp1, Wn1, bn1,
           Wpo, bpo, Wno, bno):
    f32 = jnp.float32
    src = edge_index[0]
    dst = edge_index[1]
    x = jnp.concatenate([nodes_feat, jnp.zeros((NP - N, IN_DIM), f32)], axis=0)
    zer = jnp.zeros((ZR, F), f32)

    h0, g1p = _dense_pre(x, W_emb.T, b_emb[None], Wp1.T, bp1[None])
    agg1 = _sc_spmm(g1p, src, dst, zer)
    h1, g2p = _dense_mid(h0, agg1, Wn1[:, :H].T, Wn1[:, H:].T, bn1[None],
                         Wpo.T, bpo[None])
    agg2 = _sc_spmm(g2p, src, dst, zer)
    return _dense_fin(h1, agg2, Wno[:, :H].T, Wno[:, H:].T, bno[None])
